# Initial kernel scaffold; baseline (speedup 1.0000x reference)
import jax
import jax.numpy as jnp
from jax.experimental import pallas as pl


def kernel(x_nchw, cw1, cb1, cw2, cb2, cw3, cb3, cw4, cb4, fc1w, fc1b, fc2w, fc2b, fc3w, fc3b):
    raise NotImplementedError("write your pallas kernel here")


# speedup vs baseline: 2.4748x; 2.4748x over previous
import jax
import jax.numpy as jnp
from jax.experimental import pallas as pl
from jax.experimental.pallas import tpu as pltpu

_BF16 = jnp.bfloat16
_F32 = jnp.float32

_LAYERS = (
    (128, 128, 3, 16, 16, 4),
    (64, 64, 16, 32, 8, 2),
    (32, 32, 32, 64, 4, 1),
    (16, 16, 64, 128, 2, 1),
)
_NB = 2


def _conv_layer(src, w_ref, bias, patch_ref, dst_write,
                H, W, C, Co, P, n_chunks):
    G = W // P
    PC = P * C
    WIN = (P + 2) * C
    Hc = H // n_chunks
    L = (P // 2) * Co
    for c in range(n_chunks):
        h0 = c * Hc
        for ky in range(3):
            slab = src(h0 + ky, h0 + ky + Hc)
            for g in range(G):
                patch_ref[g * Hc:(g + 1) * Hc, ky * WIN:(ky + 1) * WIN] = (
                    slab[:, g * PC:g * PC + WIN])
        y = jnp.dot(patch_ref[...], w_ref[...],
                    preferred_element_type=_F32)
        y = y.reshape(G * Hc, P // 2, 2, Co)
        y = jnp.maximum(y[:, :, 0, :], y[:, :, 1, :])
        y = y.reshape(G, Hc // 2, 2, L)
        y = jnp.maximum(y[:, :, 0, :], y[:, :, 1, :])
        y = jnp.maximum(y + bias, 0.0).astype(_BF16)
        for g in range(G):
            dst_write(h0 // 2, g, y[g])


def _conv_kernel(x_ref, w1, w2, w3, w4, b_ref, o_ref,
                 xp2, xp3, xp4, p1, p2, p3, p4):
    for ref, (Hn, Cn) in ((xp2, (64, 16)), (xp3, (32, 32)), (xp4, (16, 64))):
        hp, wc = ref.shape
        ref[0:1, :] = jnp.zeros((1, wc), _BF16)
        ref[hp - 1:hp, :] = jnp.zeros((1, wc), _BF16)
        ref[:, 0:Cn] = jnp.zeros((hp, Cn), _BF16)
        ref[:, wc - Cn:wc] = jnp.zeros((hp, Cn), _BF16)

    def mk_store(ref, Co, L):
        def w(r0, g, t):
            ref[1 + r0:1 + r0 + t.shape[0], Co + g * L:Co + (g + 1) * L] = t
        return w

    for i in range(_NB):
        _conv_layer(lambda a, b: x_ref[i, a:b, :], w1, b_ref[0:1], p1,
                    mk_store(xp2, 16, 128), *_LAYERS[0][:2], *_LAYERS[0][2:])
        _conv_layer(lambda a, b: xp2[a:b, :], w2, b_ref[1:2], p2,
                    mk_store(xp3, 32, 128), *_LAYERS[1][:2], *_LAYERS[1][2:])
        _conv_layer(lambda a, b: xp3[a:b, :], w3, b_ref[2:3], p3,
                    mk_store(xp4, 64, 128), *_LAYERS[2][:2], *_LAYERS[2][2:])

        def out_store(r0, g, t, i=i):
            o_ref[i, r0:r0 + t.shape[0], g * 128:(g + 1) * 128] = t
        _conv_layer(lambda a, b: xp4[a:b, :], w4, b_ref[3:4], p4,
                    out_store, *_LAYERS[3][:2], *_LAYERS[3][2:])


def _conv_stack(x2, w1p, w2p, w3p, w4p, b_all):
    n = x2.shape[0]
    return pl.pallas_call(
        _conv_kernel,
        out_shape=jax.ShapeDtypeStruct((n, 8, 1024), _BF16),
        grid=(n // _NB,),
        in_specs=[
            pl.BlockSpec((_NB, 130, 390), lambda i: (i, 0, 0)),
            pl.BlockSpec((162, 256), lambda i: (0, 0)),
            pl.BlockSpec((480, 256), lambda i: (0, 0)),
            pl.BlockSpec((576, 256), lambda i: (0, 0)),
            pl.BlockSpec((768, 256), lambda i: (0, 0)),
            pl.BlockSpec((4, 128), lambda i: (0, 0)),
        ],
        out_specs=pl.BlockSpec((_NB, 8, 1024), lambda i: (i, 0, 0)),
        scratch_shapes=[
            pltpu.VMEM((66, 1056), _BF16),
            pltpu.VMEM((34, 1088), _BF16),
            pltpu.VMEM((18, 1152), _BF16),
            pltpu.VMEM((256, 162), _BF16),
            pltpu.VMEM((256, 480), _BF16),
            pltpu.VMEM((256, 576), _BF16),
            pltpu.VMEM((128, 768), _BF16),
        ],
        compiler_params=pltpu.CompilerParams(
            dimension_semantics=("parallel",),
            vmem_limit_bytes=32 * 1024 * 1024),
    )(x2, w1p, w2p, w3p, w4p, b_all)


def _fc_kernel(x_ref, w1, b1, w2, b2, w3, b3, o_ref):
    h1 = jnp.dot(x_ref[...], w1[...], preferred_element_type=_F32)
    h1 = jnp.maximum(h1 + b1[...], 0.0)
    h2 = jnp.dot(h1, w2[...], preferred_element_type=_F32) + b2[...]
    o_ref[...] = jnp.dot(h2, w3[...], preferred_element_type=_F32) + b3[...]


def _fc_head(x, w1, b1, w2, b2, w3, b3):
    n, k = x.shape
    m = n // 2
    return pl.pallas_call(
        _fc_kernel,
        out_shape=jax.ShapeDtypeStruct((n, 37), _F32),
        grid=(2,),
        in_specs=[
            pl.BlockSpec((m, k), lambda i: (i, 0)),
            pl.BlockSpec((k, 256), lambda i: (0, 0)),
            pl.BlockSpec((1, 256), lambda i: (0, 0)),
            pl.BlockSpec((256, 128), lambda i: (0, 0)),
            pl.BlockSpec((1, 128), lambda i: (0, 0)),
            pl.BlockSpec((128, 37), lambda i: (0, 0)),
            pl.BlockSpec((1, 37), lambda i: (0, 0)),
        ],
        out_specs=pl.BlockSpec((m, 37), lambda i: (i, 0)),
        compiler_params=pltpu.CompilerParams(
            dimension_semantics=("parallel",),
            vmem_limit_bytes=32 * 1024 * 1024),
    )(x, w1, b1, w2, b2, w3, b3)


def _pack_w(cw, C, Co, P):
    w3 = cw.reshape(3, 3, C, Co)
    wf = jnp.zeros((3, P + 2, C, P, Co), _F32)
    for p in range(P):
        wf = wf.at[:, p:p + 3, :, p, :].set(w3)
    return wf.reshape(3 * (P + 2) * C, P * Co).astype(_BF16)


def kernel(x_nchw, cw1, cb1, cw2, cb2, cw3, cb3, cw4, cb4,
           fc1w, fc1b, fc2w, fc2b, fc3w, fc3b):
    n = x_nchw.shape[0]
    x = jnp.transpose(x_nchw, (0, 2, 3, 1))
    x = jnp.pad(x, ((0, 0), (1, 1), (1, 1), (0, 0)))
    x2 = x.reshape(n, 130, 390).astype(_BF16)

    w1p = _pack_w(cw1, 3, 16, 16)
    w2p = _pack_w(cw2, 16, 32, 8)
    w3p = _pack_w(cw3, 32, 64, 4)
    w4p = _pack_w(cw4, 64, 128, 2)
    b_all = jnp.concatenate(
        [jnp.tile(cb1, (1, 8)), jnp.tile(cb2, (1, 4)),
         jnp.tile(cb3, (1, 2)), cb4], axis=0)

    h = _conv_stack(x2, w1p, w2p, w3p, w4p, b_all)
    return _fc_head(h.reshape(n, 8192), fc1w.astype(_BF16), fc1b,
                    fc2w, fc2b, fc3w, fc3b)


# speedup vs baseline: 18.9105x; 7.6413x over previous
import jax
import jax.numpy as jnp
from jax.experimental import pallas as pl
from jax.experimental.pallas import tpu as pltpu

_BF16 = jnp.bfloat16
_F32 = jnp.float32

_LAYERS = (
    (128, 128, 3, 16, 16, 4),
    (64, 64, 16, 32, 8, 2),
    (32, 32, 32, 64, 4, 1),
    (16, 16, 64, 128, 2, 1),
)
_NB = 8


def _conv_layer(src, w_ref, bias, patch_ref, dst_write,
                H, W, C, Co, P, n_chunks, off=0):
    G = W // P
    PC = P * C
    WIN = (P + 2) * C
    Hc = H // n_chunks
    for c in range(n_chunks):
        h0 = c * Hc
        for ky in range(3):
            slab = src(h0 + ky, h0 + ky + Hc)
            for g in range(G):
                patch_ref[g * Hc:(g + 1) * Hc, ky * WIN:(ky + 1) * WIN] = (
                    slab[:, off + g * PC:off + g * PC + WIN])
        y = jnp.dot(patch_ref[...], w_ref[...],
                    preferred_element_type=_F32)
        y = y.reshape(G * Hc // 2, 512)
        y = jnp.maximum(jnp.maximum(y[:, 0:128], y[:, 128:256]),
                        jnp.maximum(y[:, 256:384], y[:, 384:512]))
        y = jnp.maximum(y + bias, 0.0).astype(_BF16)
        for g in range(G):
            dst_write(h0 // 2, g, y[g * (Hc // 2):(g + 1) * (Hc // 2), :])


def _conv_kernel(x_ref, w1, w2, w3, w4, b_ref, o_ref,
                 xp2, xp3, xp4, p1, p2, p3, p4):
    for ref, Cn in ((xp2, 16), (xp3, 32), (xp4, 64)):
        hp = ref.shape[0]
        ref[0:1, :] = jnp.zeros((1, ref.shape[1]), _BF16)
        ref[hp - 1:hp, :] = jnp.zeros((1, ref.shape[1]), _BF16)
        ref[:, 128 - Cn:128] = jnp.zeros((hp, Cn), _BF16)
        ref[:, 1152:1152 + Cn] = jnp.zeros((hp, Cn), _BF16)

    def mk_store(ref):
        def w(r0, g, t):
            ref[1 + r0:1 + r0 + t.shape[0], 128 + g * 128:256 + g * 128] = t
        return w

    for i in range(_NB):
        _conv_layer(lambda a, b: x_ref[i, a:b, :], w1, b_ref[0:1], p1,
                    mk_store(xp2), *_LAYERS[0][:2], *_LAYERS[0][2:])
        _conv_layer(lambda a, b: xp2[a:b, :], w2, b_ref[1:2], p2,
                    mk_store(xp3), *_LAYERS[1][:2], *_LAYERS[1][2:], off=112)
        _conv_layer(lambda a, b: xp3[a:b, :], w3, b_ref[2:3], p3,
                    mk_store(xp4), *_LAYERS[2][:2], *_LAYERS[2][2:], off=96)

        def out_store(r0, g, t, i=i):
            o_ref[i, r0:r0 + t.shape[0], g * 128:(g + 1) * 128] = t
        _conv_layer(lambda a, b: xp4[a:b, :], w4, b_ref[3:4], p4,
                    out_store, *_LAYERS[3][:2], *_LAYERS[3][2:], off=64)


def _conv_stack(x2, w1p, w2p, w3p, w4p, b_all):
    n = x2.shape[0]
    return pl.pallas_call(
        _conv_kernel,
        out_shape=jax.ShapeDtypeStruct((n, 8, 1024), _BF16),
        grid=(n // _NB,),
        in_specs=[
            pl.BlockSpec((_NB, 130, 390), lambda i: (i, 0, 0)),
            pl.BlockSpec((162, 256), lambda i: (0, 0)),
            pl.BlockSpec((480, 256), lambda i: (0, 0)),
            pl.BlockSpec((576, 256), lambda i: (0, 0)),
            pl.BlockSpec((768, 256), lambda i: (0, 0)),
            pl.BlockSpec((4, 128), lambda i: (0, 0)),
        ],
        out_specs=pl.BlockSpec((_NB, 8, 1024), lambda i: (i, 0, 0)),
        scratch_shapes=[
            pltpu.VMEM((66, 1168), _BF16),
            pltpu.VMEM((34, 1184), _BF16),
            pltpu.VMEM((18, 1216), _BF16),
            pltpu.VMEM((256, 162), _BF16),
            pltpu.VMEM((256, 480), _BF16),
            pltpu.VMEM((256, 576), _BF16),
            pltpu.VMEM((128, 768), _BF16),
        ],
        compiler_params=pltpu.CompilerParams(
            dimension_semantics=("parallel",),
            vmem_limit_bytes=32 * 1024 * 1024),
    )(x2, w1p, w2p, w3p, w4p, b_all)


def _fc_kernel(x_ref, w1, b1, w2, b2, w3, b3, o_ref):
    h1 = jnp.dot(x_ref[...], w1[...], preferred_element_type=_F32)
    h1 = jnp.maximum(h1 + b1[...], 0.0)
    h2 = jnp.dot(h1, w2[...], preferred_element_type=_F32) + b2[...]
    o_ref[...] = jnp.dot(h2, w3[...], preferred_element_type=_F32) + b3[...]


def _fc_head(x, w1, b1, w2, b2, w3, b3):
    n, k = x.shape
    m = n // 2
    return pl.pallas_call(
        _fc_kernel,
        out_shape=jax.ShapeDtypeStruct((n, 37), _F32),
        grid=(2,),
        in_specs=[
            pl.BlockSpec((m, k), lambda i: (i, 0)),
            pl.BlockSpec((k, 256), lambda i: (0, 0)),
            pl.BlockSpec((1, 256), lambda i: (0, 0)),
            pl.BlockSpec((256, 128), lambda i: (0, 0)),
            pl.BlockSpec((1, 128), lambda i: (0, 0)),
            pl.BlockSpec((128, 37), lambda i: (0, 0)),
            pl.BlockSpec((1, 37), lambda i: (0, 0)),
        ],
        out_specs=pl.BlockSpec((m, 37), lambda i: (i, 0)),
        compiler_params=pltpu.CompilerParams(
            dimension_semantics=("parallel",),
            vmem_limit_bytes=32 * 1024 * 1024),
    )(x, w1, b1, w2, b2, w3, b3)


def _pack_w(cw, C, Co, P):
    w3 = cw.reshape(3, 3, C, Co)
    wp = jnp.pad(w3, ((0, 0), (0, P - 1), (0, 0), (0, 0)))
    idx = (jnp.arange(P + 2)[:, None] - jnp.arange(P)[None, :]) % (P + 2)
    wf = wp[:, idx, :, :]
    wf = jnp.transpose(wf, (0, 1, 3, 2, 4))
    s = wf.shape
    wf = wf.reshape(s[0], s[1], s[2], P // 2, 2, Co)
    wf = jnp.transpose(wf, (0, 1, 2, 4, 3, 5))
    return wf.reshape(3 * (P + 2) * C, P * Co).astype(_BF16)


def kernel(x_nchw, cw1, cb1, cw2, cb2, cw3, cb3, cw4, cb4,
           fc1w, fc1b, fc2w, fc2b, fc3w, fc3b):
    n = x_nchw.shape[0]
    x = jnp.transpose(x_nchw, (0, 2, 3, 1))
    x = jnp.pad(x, ((0, 0), (1, 1), (1, 1), (0, 0)))
    x2 = x.reshape(n, 130, 390).astype(_BF16)

    w1p = _pack_w(cw1, 3, 16, 16)
    w2p = _pack_w(cw2, 16, 32, 8)
    w3p = _pack_w(cw3, 32, 64, 4)
    w4p = _pack_w(cw4, 64, 128, 2)
    b_all = jnp.concatenate(
        [jnp.tile(cb1, (1, 8)), jnp.tile(cb2, (1, 4)),
         jnp.tile(cb3, (1, 2)), cb4], axis=0)

    h = _conv_stack(x2, w1p, w2p, w3p, w4p, b_all)
    return _fc_head(h.reshape(n, 8192), fc1w.astype(_BF16), fc1b,
                    fc2w, fc2b, fc3w, fc3b)


# speedup vs baseline: 19.4162x; 1.0267x over previous
import jax
import jax.numpy as jnp
from jax.experimental import pallas as pl
from jax.experimental.pallas import tpu as pltpu

_BF16 = jnp.bfloat16
_F32 = jnp.float32

_LAYERS = (
    (128, 128, 3, 16, 16, 2),
    (64, 64, 16, 32, 8, 1),
    (32, 32, 32, 64, 4, 1),
    (16, 16, 64, 128, 2, 1),
)
_NB = 8


def _conv_layer(src, w_ref, bias, patch_ref, dst_write,
                H, W, C, Co, P, n_chunks, off=0):
    G = W // P
    PC = P * C
    WIN = (P + 2) * C
    Hc = H // n_chunks
    for c in range(n_chunks):
        h0 = c * Hc
        for ky in range(3):
            slab = src(h0 + ky, h0 + ky + Hc)
            for g in range(G):
                patch_ref[g * Hc:(g + 1) * Hc, ky * WIN:(ky + 1) * WIN] = (
                    slab[:, off + g * PC:off + g * PC + WIN])
        y = jnp.dot(patch_ref[...], w_ref[...],
                    preferred_element_type=_F32)
        y = y.reshape(G * Hc // 2, 512)
        y = jnp.maximum(jnp.maximum(y[:, 0:128], y[:, 128:256]),
                        jnp.maximum(y[:, 256:384], y[:, 384:512]))
        y = jnp.maximum(y + bias, 0.0).astype(_BF16)
        for g in range(G):
            dst_write(h0 // 2, g, y[g * (Hc // 2):(g + 1) * (Hc // 2), :])


def _conv_kernel(x_ref, w1, w2, w3, w4, b_ref, o_ref,
                 xp2, xp3, xp4, p1, p2, p3, p4):
    for ref, Cn in ((xp2, 16), (xp3, 32), (xp4, 64)):
        hp = ref.shape[0]
        ref[0:1, :] = jnp.zeros((1, ref.shape[1]), _BF16)
        ref[hp - 1:hp, :] = jnp.zeros((1, ref.shape[1]), _BF16)
        ref[:, 128 - Cn:128] = jnp.zeros((hp, Cn), _BF16)
        ref[:, 1152:1152 + Cn] = jnp.zeros((hp, Cn), _BF16)

    def mk_store(ref):
        def w(r0, g, t):
            ref[1 + r0:1 + r0 + t.shape[0], 128 + g * 128:256 + g * 128] = t
        return w

    for i in range(_NB):
        _conv_layer(lambda a, b: x_ref[i, a:b, :], w1, b_ref[0:1], p1,
                    mk_store(xp2), *_LAYERS[0][:2], *_LAYERS[0][2:])
        _conv_layer(lambda a, b: xp2[a:b, :], w2, b_ref[1:2], p2,
                    mk_store(xp3), *_LAYERS[1][:2], *_LAYERS[1][2:], off=112)
        _conv_layer(lambda a, b: xp3[a:b, :], w3, b_ref[2:3], p3,
                    mk_store(xp4), *_LAYERS[2][:2], *_LAYERS[2][2:], off=96)

        def out_store(r0, g, t, i=i):
            o_ref[i, r0:r0 + t.shape[0], g * 128:(g + 1) * 128] = t
        _conv_layer(lambda a, b: xp4[a:b, :], w4, b_ref[3:4], p4,
                    out_store, *_LAYERS[3][:2], *_LAYERS[3][2:], off=64)


def _conv_stack(x2, w1p, w2p, w3p, w4p, b_all):
    n = x2.shape[0]
    return pl.pallas_call(
        _conv_kernel,
        out_shape=jax.ShapeDtypeStruct((n, 8, 1024), _BF16),
        grid=(n // _NB,),
        in_specs=[
            pl.BlockSpec((_NB, 130, 390), lambda i: (i, 0, 0)),
            pl.BlockSpec((162, 256), lambda i: (0, 0)),
            pl.BlockSpec((480, 256), lambda i: (0, 0)),
            pl.BlockSpec((576, 256), lambda i: (0, 0)),
            pl.BlockSpec((768, 256), lambda i: (0, 0)),
            pl.BlockSpec((4, 128), lambda i: (0, 0)),
        ],
        out_specs=pl.BlockSpec((_NB, 8, 1024), lambda i: (i, 0, 0)),
        scratch_shapes=[
            pltpu.VMEM((66, 1168), _BF16),
            pltpu.VMEM((34, 1184), _BF16),
            pltpu.VMEM((18, 1216), _BF16),
            pltpu.VMEM((512, 162), _BF16),
            pltpu.VMEM((512, 480), _BF16),
            pltpu.VMEM((256, 576), _BF16),
            pltpu.VMEM((128, 768), _BF16),
        ],
        compiler_params=pltpu.CompilerParams(
            dimension_semantics=("parallel",),
            vmem_limit_bytes=32 * 1024 * 1024),
    )(x2, w1p, w2p, w3p, w4p, b_all)


def _fc_kernel(x_ref, w1, b1, w2, b2, w3, b3, o_ref):
    h1 = jnp.dot(x_ref[...], w1[...], preferred_element_type=_F32)
    h1 = jnp.maximum(h1 + b1[...], 0.0)
    h2 = jnp.dot(h1, w2[...], preferred_element_type=_F32) + b2[...]
    o_ref[...] = jnp.dot(h2, w3[...], preferred_element_type=_F32) + b3[...]


def _fc_head(x, w1, b1, w2, b2, w3, b3):
    n, k = x.shape
    m = n // 2
    return pl.pallas_call(
        _fc_kernel,
        out_shape=jax.ShapeDtypeStruct((n, 37), _F32),
        grid=(2,),
        in_specs=[
            pl.BlockSpec((m, k), lambda i: (i, 0)),
            pl.BlockSpec((k, 256), lambda i: (0, 0)),
            pl.BlockSpec((1, 256), lambda i: (0, 0)),
            pl.BlockSpec((256, 128), lambda i: (0, 0)),
            pl.BlockSpec((1, 128), lambda i: (0, 0)),
            pl.BlockSpec((128, 37), lambda i: (0, 0)),
            pl.BlockSpec((1, 37), lambda i: (0, 0)),
        ],
        out_specs=pl.BlockSpec((m, 37), lambda i: (i, 0)),
        compiler_params=pltpu.CompilerParams(
            dimension_semantics=("parallel",),
            vmem_limit_bytes=32 * 1024 * 1024),
    )(x, w1, b1, w2, b2, w3, b3)


def _pack_w(cw, C, Co, P):
    w3 = cw.reshape(3, 3, C, Co)
    wp = jnp.pad(w3, ((0, 0), (0, P - 1), (0, 0), (0, 0)))
    idx = (jnp.arange(P + 2)[:, None] - jnp.arange(P)[None, :]) % (P + 2)
    wf = wp[:, idx, :, :]
    wf = jnp.transpose(wf, (0, 1, 3, 2, 4))
    s = wf.shape
    wf = wf.reshape(s[0], s[1], s[2], P // 2, 2, Co)
    wf = jnp.transpose(wf, (0, 1, 2, 4, 3, 5))
    return wf.reshape(3 * (P + 2) * C, P * Co).astype(_BF16)


def kernel(x_nchw, cw1, cb1, cw2, cb2, cw3, cb3, cw4, cb4,
           fc1w, fc1b, fc2w, fc2b, fc3w, fc3b):
    n = x_nchw.shape[0]
    x = jnp.transpose(x_nchw, (0, 2, 3, 1))
    x = jnp.pad(x, ((0, 0), (1, 1), (1, 1), (0, 0)))
    x2 = x.reshape(n, 130, 390).astype(_BF16)

    w1p = _pack_w(cw1, 3, 16, 16)
    w2p = _pack_w(cw2, 16, 32, 8)
    w3p = _pack_w(cw3, 32, 64, 4)
    w4p = _pack_w(cw4, 64, 128, 2)
    b_all = jnp.concatenate(
        [jnp.tile(cb1, (1, 8)), jnp.tile(cb2, (1, 4)),
         jnp.tile(cb3, (1, 2)), cb4], axis=0)

    h = _conv_stack(x2, w1p, w2p, w3p, w4p, b_all)
    return _fc_head(h.reshape(n, 8192), fc1w.astype(_BF16), fc1b,
                    fc2w, fc2b, fc3w, fc3b)
